# baseline (speedup 1.0000x reference)
import functools

import jax
import jax.numpy as jnp
from jax import lax
from jax.experimental import pallas as pl
from jax.experimental.pallas import tpu as pltpu

B, NB, PL, H, F = 16, 64, 24, 1024, 4096
M = B * NB
BM = 128
NSTEP = M // BM
BF = F // NSTEP


def _fused_body(vis_ref, lang_ref, invlen_ref, wv_ref, wl_hbm, bv_hbm, bl_hbm,
                lmap_ref, vmap_ref, wl_v, bv_v, bl_v, sem_wl, sem_bv, sem_bl):
    i = pl.program_id(0)

    @pl.when(i == 0)
    def _():
        pltpu.async_copy(wl_hbm, wl_v, sem_wl)
        pltpu.async_copy(bv_hbm, bv_v, sem_bv)
        pltpu.async_copy(bl_hbm, bl_v, sem_bl)

    inv_len = 1.0 / invlen_ref[...].astype(jnp.float32)
    pooled = jnp.sum(lang_ref[...], axis=1) * inv_len

    prod = lax.dot_general(vis_ref[...], wv_ref[...], (((1,), (1,)), ((), ())),
                           preferred_element_type=jnp.float32)

    @pl.when(i == 0)
    def _():
        pltpu.make_async_copy(wl_hbm, wl_v, sem_wl).wait()
        pltpu.make_async_copy(bv_hbm, bv_v, sem_bv).wait()
        pltpu.make_async_copy(bl_hbm, bl_v, sem_bl).wait()
        vmap_ref[...] = prod + bv_v[...]

    @pl.when(i > 0)
    def _():
        vmap_ref[...] += prod

    lmap_ref[...] = (
        lax.dot_general(pooled, wl_v[...], (((1,), (1,)), ((), ())),
                        preferred_element_type=jnp.float32)
        + bl_v[...]
    )


@functools.partial(jax.jit, static_argnames=())
def kernel(vision, language, phrase_lengths, Wv, bv, Wl, bl):
    vis = vision.reshape(M, F)
    lang = language.reshape(M, PL, H)

    lmap, vmap = pl.pallas_call(
        _fused_body,
        grid=(NSTEP,),
        in_specs=[
            pl.BlockSpec((M, BF), lambda i: (0, i)),
            pl.BlockSpec((BM, PL, H), lambda i: (i, 0, 0)),
            pl.BlockSpec((BM, 1), lambda i: (i, 0)),
            pl.BlockSpec((H, BF), lambda i: (0, i)),
            pl.BlockSpec(memory_space=pl.ANY),
            pl.BlockSpec(memory_space=pl.ANY),
            pl.BlockSpec(memory_space=pl.ANY),
        ],
        out_specs=[
            pl.BlockSpec((BM, H), lambda i: (i, 0)),
            pl.BlockSpec((M, H), lambda i: (0, 0)),
        ],
        out_shape=[
            jax.ShapeDtypeStruct((M, H), jnp.float32),
            jax.ShapeDtypeStruct((M, H), jnp.float32),
        ],
        scratch_shapes=[
            pltpu.VMEM((H, H), jnp.float32),
            pltpu.VMEM((1, H), jnp.float32),
            pltpu.VMEM((1, H), jnp.float32),
            pltpu.SemaphoreType.DMA,
            pltpu.SemaphoreType.DMA,
            pltpu.SemaphoreType.DMA,
        ],
    )(vis, lang, phrase_lengths.reshape(M, 1), Wv, Wl, bv.reshape(1, H), bl.reshape(1, H))

    return (lmap.reshape(B, NB, H), vmap.reshape(B, NB, H))
